# baseline (speedup 1.0000x reference)
import functools

import jax
import jax.numpy as jnp
from jax import lax
from jax.experimental import pallas as pl
from jax.experimental.pallas import tpu as pltpu
from jax.experimental.pallas import tpu_sc as plsc

N = 10000
E = 320000
IN_DIM = 128
HID = 64
ALIGN = 32
NUM_CLASSES = 7

NC = 2
NS = 16
NW = NC * NS
CH = 128
NCHUNK = E // CH
BASE = NCHUNK // NW
EXTRA = NCHUNK - BASE * NW
CMAX = BASE + 1
SROWS = 10240
RPT = SROWS // NS

BR = 2048
GRID = SROWS // BR

MAGIC = 0x5F3759DF

_mesh = plsc.VectorSubcoreMesh(core_axis_name="c", subcore_axis_name="s")
_sc_params = pltpu.CompilerParams(use_tc_tiling_on_sc=False,
                                  needs_layout_passes=False)


def _edge_slab(edge_hbm, idx_v, row_start, w):
    pltpu.sync_copy(edge_hbm.at[pl.ds(row_start, BASE)],
                    idx_v.at[pl.ds(0, BASE)])

    @pl.when(w < EXTRA)
    def _():
        pltpu.sync_copy(edge_hbm.at[pl.ds(row_start + BASE, 1)],
                        idx_v.at[pl.ds(BASE, 1)])


@functools.partial(
    pl.kernel,
    out_type=jax.ShapeDtypeStruct((NC, SROWS, 16), jnp.float32),
    mesh=_mesh,
    scratch_types=[
        pltpu.VMEM((CMAX, CH), jnp.int32),
        pltpu.VMEM((CH, 16), jnp.float32),
        pltpu.VMEM_SHARED((SROWS, 16), jnp.float32),
    ],
    compiler_params=_sc_params,
)
def _deg_kernel(dst_hbm, zeros_hbm, ones_hbm, deg_out, dst_v, ones_v, acc_sh):
    cid = lax.axis_index("c")
    sid = lax.axis_index("s")
    w = cid * NS + sid
    row_start = w * BASE + jnp.minimum(w, EXTRA)
    ncap = BASE + jnp.where(w < EXTRA, 1, 0)

    _edge_slab(dst_hbm, dst_v, row_start, w)
    pltpu.sync_copy(ones_hbm, ones_v)
    pltpu.sync_copy(zeros_hbm, acc_sh.at[pl.ds(sid * RPT, RPT)])
    plsc.subcore_barrier()

    def chunk(j, _):
        pltpu.sync_copy(ones_v, acc_sh.at[dst_v.at[j]], add=True)
        return ()

    lax.fori_loop(0, ncap, chunk, ())
    plsc.subcore_barrier()
    pltpu.sync_copy(acc_sh.at[pl.ds(sid * RPT, RPT)],
                    deg_out.at[cid, pl.ds(sid * RPT, RPT)])


@functools.partial(
    pl.kernel,
    out_type=jax.ShapeDtypeStruct((NC, SROWS, HID), jnp.float32),
    mesh=_mesh,
    scratch_types=[
        pltpu.VMEM((CMAX, CH), jnp.int32),
        pltpu.VMEM((CMAX, CH), jnp.int32),
        pltpu.VMEM((CH, 16), jnp.float32),
        pltpu.VMEM((CH, 16), jnp.float32),
        pltpu.VMEM((CH, HID), jnp.float32),
        pltpu.VMEM((CH, HID), jnp.float32),
        pltpu.SemaphoreType.DMA,
        pltpu.SemaphoreType.DMA,
        pltpu.VMEM_SHARED((SROWS, HID), jnp.float32),
        pltpu.VMEM_SHARED((SROWS, HID), jnp.float32),
    ],
    compiler_params=_sc_params,
)
def _msg_kernel(src_hbm, dst_hbm, xw_hbm, deg_hbm, zeros_hbm, p_out,
                src_v, dst_v, dpb, dpb2, buf0, buf1, sem0, sem1,
                acc_sh, y_sh):
    cid = lax.axis_index("c")
    sid = lax.axis_index("s")
    w = cid * NS + sid
    row_start = w * BASE + jnp.minimum(w, EXTRA)
    ncap = BASE + jnp.where(w < EXTRA, 1, 0)

    _edge_slab(src_hbm, src_v, row_start, w)
    _edge_slab(dst_hbm, dst_v, row_start, w)
    base = sid * RPT

    def dinv_row(r):
        d = dpb[r, :] + dpb2[r, :] + 1.0
        i = plsc.bitcast(d, jnp.int32)
        i = MAGIC - lax.shift_right_logical(i, 1)
        rr = plsc.bitcast(i, jnp.float32)
        dh = d * 0.5
        rr = rr * (1.5 - dh * rr * rr)
        rr = rr * (1.5 - dh * rr * rr)
        rr = rr * (1.5 - dh * rr * rr)
        return rr

    for k in range(RPT // CH):
        off = base + k * CH
        pltpu.sync_copy(xw_hbm.at[pl.ds(off, CH)], buf0)
        pltpu.sync_copy(deg_hbm.at[0, pl.ds(off, CH)], dpb)
        pltpu.sync_copy(deg_hbm.at[1, pl.ds(off, CH)], dpb2)

        def yrow(r, _):
            rr = dinv_row(r)
            for c in range(HID // 16):
                sl = pl.ds(c * 16, 16)
                buf0[r, sl] = buf0[r, sl] * rr
            return ()

        lax.fori_loop(0, CH, yrow, ())
        pltpu.sync_copy(buf0, y_sh.at[pl.ds(off, CH)])

    pltpu.sync_copy(zeros_hbm, acc_sh.at[pl.ds(base, RPT)])
    plsc.subcore_barrier()

    pltpu.async_copy(y_sh.at[src_v.at[0]], buf0, sem0)
    pltpu.async_copy(y_sh.at[src_v.at[1]], buf1, sem1)

    def pair(i, _):
        j0 = i * 2
        for b, (buf, sem) in enumerate(((buf0, sem0), (buf1, sem1))):
            j = j0 + b
            pltpu.make_async_copy(y_sh.at[src_v.at[j]], buf, sem).wait()
            pltpu.sync_copy(buf, acc_sh.at[dst_v.at[j]], add=True)

            @pl.when(j + 2 < ncap)
            def _():
                pltpu.async_copy(y_sh.at[src_v.at[j + 2]], buf, sem)

        return ()

    lax.fori_loop(0, BASE // 2, pair, ())

    @pl.when(ncap > BASE)
    def _():
        pltpu.make_async_copy(y_sh.at[src_v.at[BASE]], buf0, sem0).wait()
        pltpu.sync_copy(buf0, acc_sh.at[dst_v.at[BASE]], add=True)

    plsc.subcore_barrier()

    flag = jnp.where(cid == 0, 1.0, 0.0)
    for k in range(RPT // CH):
        off = base + k * CH
        pltpu.sync_copy(acc_sh.at[pl.ds(off, CH)], buf0)
        pltpu.sync_copy(y_sh.at[pl.ds(off, CH)], buf1)
        pltpu.sync_copy(deg_hbm.at[0, pl.ds(off, CH)], dpb)
        pltpu.sync_copy(deg_hbm.at[1, pl.ds(off, CH)], dpb2)

        def srow(r, _):
            rr = dinv_row(r)
            for c in range(HID // 16):
                sl = pl.ds(c * 16, 16)
                buf0[r, sl] = (buf0[r, sl] + flag * buf1[r, sl]) * rr
            return ()

        lax.fori_loop(0, CH, srow, ())
        pltpu.sync_copy(buf0, p_out.at[cid, pl.ds(off, CH)])


def _proj_body(x_ref, wg_ref, wps_ref, bps_ref, xw_ref, zsem_ref):
    x = x_ref[...]
    xw_ref[...] = jnp.dot(x, wg_ref[...], preferred_element_type=jnp.float32)
    zsem_ref[...] = (
        jnp.dot(x, wps_ref[...], preferred_element_type=jnp.float32)
        + bps_ref[...]
    )


def _proj(x, W_gcn, W_ps, b_ps):
    return pl.pallas_call(
        _proj_body,
        grid=(GRID,),
        in_specs=[
            pl.BlockSpec((BR, IN_DIM), lambda i: (i, 0)),
            pl.BlockSpec((IN_DIM, HID), lambda i: (0, 0)),
            pl.BlockSpec((IN_DIM, ALIGN), lambda i: (0, 0)),
            pl.BlockSpec((1, ALIGN), lambda i: (0, 0)),
        ],
        out_specs=[
            pl.BlockSpec((BR, HID), lambda i: (i, 0)),
            pl.BlockSpec((BR, ALIGN), lambda i: (i, 0)),
        ],
        out_shape=[
            jax.ShapeDtypeStruct((SROWS, HID), jnp.float32),
            jax.ShapeDtypeStruct((N, ALIGN), jnp.float32),
        ],
    )(x, W_gcn, W_ps, b_ps)


def _head_body(p_ref, zsem_ref, bg_ref, wpt_ref, bpt_ref,
               wcls_ref, bcls_ref, logits_ref, an_ref, ztopo_ref):
    h = jnp.maximum(p_ref[0] + p_ref[1] + bg_ref[...], 0.0)
    z_topo = (
        jnp.dot(h, wpt_ref[...], preferred_element_type=jnp.float32)
        + bpt_ref[...]
    )
    logits_ref[...] = (
        jnp.dot(z_topo, wcls_ref[...], preferred_element_type=jnp.float32)
        + bcls_ref[...]
    )
    diff = z_topo - zsem_ref[...]
    an_ref[...] = jnp.sqrt(jnp.sum(diff * diff, axis=1))
    ztopo_ref[...] = z_topo


def _heads(p_parts, z_sem, b_gcn, W_pt, b_pt, W_cls, b_cls):
    return pl.pallas_call(
        _head_body,
        grid=(GRID,),
        in_specs=[
            pl.BlockSpec((2, BR, HID), lambda i: (0, i, 0)),
            pl.BlockSpec((BR, ALIGN), lambda i: (i, 0)),
            pl.BlockSpec((1, HID), lambda i: (0, 0)),
            pl.BlockSpec((HID, ALIGN), lambda i: (0, 0)),
            pl.BlockSpec((1, ALIGN), lambda i: (0, 0)),
            pl.BlockSpec((ALIGN, NUM_CLASSES), lambda i: (0, 0)),
            pl.BlockSpec((1, NUM_CLASSES), lambda i: (0, 0)),
        ],
        out_specs=[
            pl.BlockSpec((BR, NUM_CLASSES), lambda i: (i, 0)),
            pl.BlockSpec((BR,), lambda i: (i,)),
            pl.BlockSpec((BR, ALIGN), lambda i: (i, 0)),
        ],
        out_shape=[
            jax.ShapeDtypeStruct((N, NUM_CLASSES), jnp.float32),
            jax.ShapeDtypeStruct((N,), jnp.float32),
            jax.ShapeDtypeStruct((N, ALIGN), jnp.float32),
        ],
    )(p_parts, z_sem, b_gcn, W_pt, b_pt, W_cls, b_cls)


def kernel(x, edge_index, W_gcn, b_gcn, W_pt, b_pt, W_ps, b_ps, W_cls, b_cls):
    src_r = edge_index[0].reshape(NCHUNK, CH)
    dst_r = edge_index[1].reshape(NCHUNK, CH)

    zeros16 = jnp.zeros((RPT, 16), jnp.float32)
    zeros64 = jnp.zeros((RPT, HID), jnp.float32)
    ones16 = jnp.ones((CH, 16), jnp.float32)

    deg_parts = _deg_kernel(dst_r, zeros16, ones16)
    xw, z_sem = _proj(x, W_gcn, W_ps, b_ps.reshape(1, ALIGN))
    p_parts = _msg_kernel(src_r, dst_r, xw, deg_parts, zeros64)
    logits, anomaly, z_topo = _heads(
        p_parts, z_sem, b_gcn.reshape(1, HID), W_pt,
        b_pt.reshape(1, ALIGN), W_cls, b_cls.reshape(1, NUM_CLASSES))
    return (logits, anomaly, z_topo, z_sem)


# speedup vs baseline: 1.2224x; 1.2224x over previous
import functools

import jax
import jax.numpy as jnp
from jax import lax
from jax.experimental import pallas as pl
from jax.experimental.pallas import tpu as pltpu
from jax.experimental.pallas import tpu_sc as plsc

N = 10000
E = 320000
IN_DIM = 128
HID = 64
ALIGN = 32
NUM_CLASSES = 7

NC = 2
NS = 16
NW = NC * NS
CH = 128
NCHUNK = E // CH
BASE = NCHUNK // NW
EXTRA = NCHUNK - BASE * NW
CMAX = BASE + 1
SROWS = 10240
RPT = SROWS // NS

BR = 2048
GRID = (N + BR - 1) // BR

_mesh = plsc.VectorSubcoreMesh(core_axis_name="c", subcore_axis_name="s")
_sc_params = pltpu.CompilerParams(use_tc_tiling_on_sc=False)


def _worker_bounds(cid, sid):
    w = cid * NS + sid
    row_start = w * BASE + jnp.minimum(w, EXTRA)
    ncap = BASE + jnp.where(w < EXTRA, 1, 0)
    return w, row_start, ncap


def _dst_slab(dst_hbm, dst_v, row_start, w):
    pltpu.sync_copy(dst_hbm.at[pl.ds(row_start, BASE)],
                    dst_v.at[pl.ds(0, BASE)])

    @pl.when(w < EXTRA)
    def _():
        pltpu.sync_copy(dst_hbm.at[pl.ds(row_start + BASE, 1)],
                        dst_v.at[pl.ds(BASE, 1)])


@functools.partial(
    pl.kernel,
    out_type=jax.ShapeDtypeStruct((NC, SROWS, 16), jnp.float32),
    mesh=_mesh,
    scratch_types=[
        pltpu.VMEM((CMAX, CH), jnp.int32),
        pltpu.VMEM((CH, 16), jnp.float32),
        pltpu.VMEM_SHARED((SROWS, 16), jnp.float32),
    ],
    compiler_params=_sc_params,
)
def _deg_kernel(dst_hbm, zeros_hbm, ones_hbm, deg_out, dst_v, ones_v, acc_sh):
    cid = lax.axis_index("c")
    sid = lax.axis_index("s")
    w, row_start, ncap = _worker_bounds(cid, sid)
    _dst_slab(dst_hbm, dst_v, row_start, w)
    pltpu.sync_copy(ones_hbm, ones_v)

    pltpu.sync_copy(zeros_hbm, acc_sh.at[pl.ds(sid * RPT, RPT)])
    plsc.subcore_barrier()

    def chunk(j, _):
        pltpu.sync_copy(ones_v, acc_sh.at[dst_v.at[j]], add=True)
        return ()

    lax.fori_loop(0, ncap, chunk, ())
    plsc.subcore_barrier()
    pltpu.sync_copy(acc_sh.at[pl.ds(sid * RPT, RPT)],
                    deg_out.at[cid, pl.ds(sid * RPT, RPT)])


@functools.partial(
    pl.kernel,
    out_type=jax.ShapeDtypeStruct((NC, SROWS, HID), jnp.float32),
    mesh=_mesh,
    scratch_types=[
        pltpu.VMEM((CMAX * CH,), jnp.int32),
        pltpu.VMEM((CMAX, CH), jnp.int32),
        pltpu.VMEM((CH, HID), jnp.float32),
        pltpu.VMEM((CH, HID), jnp.float32),
        pltpu.SemaphoreType.DMA,
        pltpu.SemaphoreType.DMA,
        pltpu.VMEM_SHARED((SROWS, HID), jnp.float32),
        pltpu.VMEM_SHARED((SROWS, HID), jnp.float32),
    ],
    compiler_params=_sc_params,
)
def _msg_kernel(src_hbm, dst_hbm, y_hbm, zeros_hbm, s_out,
                src_v, dst_v, buf0, buf1, sem0, sem1, acc_sh, y_sh):
    cid = lax.axis_index("c")
    sid = lax.axis_index("s")
    w, row_start, ncap = _worker_bounds(cid, sid)
    pltpu.sync_copy(src_hbm.at[pl.ds(row_start * CH, BASE * CH)],
                    src_v.at[pl.ds(0, BASE * CH)])

    @pl.when(w < EXTRA)
    def _():
        pltpu.sync_copy(src_hbm.at[pl.ds((row_start + BASE) * CH, CH)],
                        src_v.at[pl.ds(BASE * CH, CH)])

    _dst_slab(dst_hbm, dst_v, row_start, w)

    pltpu.sync_copy(y_hbm.at[pl.ds(sid * RPT, RPT)],
                    y_sh.at[pl.ds(sid * RPT, RPT)])
    pltpu.sync_copy(zeros_hbm, acc_sh.at[pl.ds(sid * RPT, RPT)])
    plsc.subcore_barrier()

    def src_idx(j):
        return src_v.at[pl.ds(j * CH, CH)]

    pltpu.async_copy(y_sh.at[src_idx(0)], buf0, sem0)
    pltpu.async_copy(y_sh.at[src_idx(1)], buf1, sem1)

    def pair(i, _):
        j0 = i * 2
        for b, (buf, sem) in enumerate(((buf0, sem0), (buf1, sem1))):
            j = j0 + b
            pltpu.make_async_copy(y_sh.at[src_idx(j)], buf, sem).wait()
            pltpu.sync_copy(buf, acc_sh.at[dst_v.at[j]], add=True)

            @pl.when(j + 2 < ncap)
            def _():
                pltpu.async_copy(y_sh.at[src_idx(j + 2)], buf, sem)

        return ()

    lax.fori_loop(0, BASE // 2, pair, ())

    @pl.when(ncap > BASE)
    def _():
        pltpu.make_async_copy(y_sh.at[src_idx(BASE)], buf0, sem0).wait()
        pltpu.sync_copy(buf0, acc_sh.at[dst_v.at[BASE]], add=True)

    plsc.subcore_barrier()
    pltpu.sync_copy(acc_sh.at[pl.ds(sid * RPT, RPT)],
                    s_out.at[cid, pl.ds(sid * RPT, RPT)])


def _proj_body(x_ref, wg_ref, wps_ref, bps_ref, xw_ref, zsem_ref):
    x = x_ref[...]
    xw_ref[...] = jnp.dot(x, wg_ref[...], preferred_element_type=jnp.float32)
    zsem_ref[...] = (
        jnp.dot(x, wps_ref[...], preferred_element_type=jnp.float32)
        + bps_ref[...]
    )


def _proj(x, W_gcn, W_ps, b_ps):
    return pl.pallas_call(
        _proj_body,
        grid=(GRID,),
        in_specs=[
            pl.BlockSpec((BR, IN_DIM), lambda i: (i, 0)),
            pl.BlockSpec((IN_DIM, HID), lambda i: (0, 0)),
            pl.BlockSpec((IN_DIM, ALIGN), lambda i: (0, 0)),
            pl.BlockSpec((1, ALIGN), lambda i: (0, 0)),
        ],
        out_specs=[
            pl.BlockSpec((BR, HID), lambda i: (i, 0)),
            pl.BlockSpec((BR, ALIGN), lambda i: (i, 0)),
        ],
        out_shape=[
            jax.ShapeDtypeStruct((N, HID), jnp.float32),
            jax.ShapeDtypeStruct((N, ALIGN), jnp.float32),
        ],
    )(x, W_gcn, W_ps, b_ps)


def _scale_body(dp_ref, xw_ref, y_ref, dinv_ref):
    deg = dp_ref[0, :, 0:1] + dp_ref[1, :, 0:1] + 1.0
    dinv = lax.rsqrt(deg)
    y_ref[...] = dinv * xw_ref[...]
    dinv_ref[...] = jnp.broadcast_to(dinv, dinv_ref.shape)


def _scale(deg_parts, xw):
    return pl.pallas_call(
        _scale_body,
        grid=(GRID,),
        in_specs=[
            pl.BlockSpec((2, BR, 16), lambda i: (0, i, 0)),
            pl.BlockSpec((BR, HID), lambda i: (i, 0)),
        ],
        out_specs=[
            pl.BlockSpec((BR, HID), lambda i: (i, 0)),
            pl.BlockSpec((BR, 16), lambda i: (i, 0)),
        ],
        out_shape=[
            jax.ShapeDtypeStruct((SROWS, HID), jnp.float32),
            jax.ShapeDtypeStruct((N, 16), jnp.float32),
        ],
    )(deg_parts, xw)


def _head_body(s_ref, y_ref, dinv_ref, zsem_ref, bg_ref, wpt_ref, bpt_ref,
               wcls_ref, bcls_ref, logits_ref, an_ref, ztopo_ref):
    dinv = dinv_ref[:, 0:1]
    s_tot = s_ref[0] + s_ref[1] + y_ref[...]
    h = jnp.maximum(dinv * s_tot + bg_ref[...], 0.0)
    z_topo = (
        jnp.dot(h, wpt_ref[...], preferred_element_type=jnp.float32)
        + bpt_ref[...]
    )
    logits_ref[...] = (
        jnp.dot(z_topo, wcls_ref[...], preferred_element_type=jnp.float32)
        + bcls_ref[...]
    )
    diff = z_topo - zsem_ref[...]
    an_ref[...] = jnp.sqrt(jnp.sum(diff * diff, axis=1))
    ztopo_ref[...] = z_topo


def _heads(s_parts, y, dinv, z_sem, b_gcn, W_pt, b_pt, W_cls, b_cls):
    return pl.pallas_call(
        _head_body,
        grid=(GRID,),
        in_specs=[
            pl.BlockSpec((2, BR, HID), lambda i: (0, i, 0)),
            pl.BlockSpec((BR, HID), lambda i: (i, 0)),
            pl.BlockSpec((BR, 16), lambda i: (i, 0)),
            pl.BlockSpec((BR, ALIGN), lambda i: (i, 0)),
            pl.BlockSpec((1, HID), lambda i: (0, 0)),
            pl.BlockSpec((HID, ALIGN), lambda i: (0, 0)),
            pl.BlockSpec((1, ALIGN), lambda i: (0, 0)),
            pl.BlockSpec((ALIGN, NUM_CLASSES), lambda i: (0, 0)),
            pl.BlockSpec((1, NUM_CLASSES), lambda i: (0, 0)),
        ],
        out_specs=[
            pl.BlockSpec((BR, NUM_CLASSES), lambda i: (i, 0)),
            pl.BlockSpec((BR,), lambda i: (i,)),
            pl.BlockSpec((BR, ALIGN), lambda i: (i, 0)),
        ],
        out_shape=[
            jax.ShapeDtypeStruct((N, NUM_CLASSES), jnp.float32),
            jax.ShapeDtypeStruct((N,), jnp.float32),
            jax.ShapeDtypeStruct((N, ALIGN), jnp.float32),
        ],
    )(s_parts, y, dinv, z_sem, b_gcn, W_pt, b_pt, W_cls, b_cls)


def kernel(x, edge_index, W_gcn, b_gcn, W_pt, b_pt, W_ps, b_ps, W_cls, b_cls):
    src_r = edge_index[0]
    dst_r = edge_index[1].reshape(NCHUNK, CH)

    zeros16 = jnp.zeros((RPT, 16), jnp.float32)
    zeros64 = jnp.zeros((RPT, HID), jnp.float32)
    ones16 = jnp.ones((CH, 16), jnp.float32)

    deg_parts = _deg_kernel(dst_r, zeros16, ones16)
    xw, z_sem = _proj(x, W_gcn, W_ps, b_ps.reshape(1, ALIGN))
    y, dinv = _scale(deg_parts, xw)
    s_parts = _msg_kernel(src_r, dst_r, y, zeros64)
    logits, anomaly, z_topo = _heads(
        s_parts, y, dinv, z_sem, b_gcn.reshape(1, HID), W_pt,
        b_pt.reshape(1, ALIGN), W_cls, b_cls.reshape(1, NUM_CLASSES))
    return (logits, anomaly, z_topo, z_sem)
